# baseline (speedup 1.0000x reference)
import jax
import jax.numpy as jnp
from jax import lax
from jax.experimental import pallas as pl
from jax.experimental.pallas import tpu as pltpu
from jax.experimental.pallas import tpu_sc as plsc

N_COMPONENT = 64
H_HOP = 2
LAYER_L = 2
DEG = 32

N = 10000
KDIM = 64

NW = 32
FPT = KDIM // NW
SEG = 400
SEGS = N // SEG
BLOCKS = SEG // 16


def _make_layer():
    def body(adj_hbm, t_hbm, out_hbm, tab_v, h1_v, idx0_v, idx1_v,
             sem0, sem1):
        cid = lax.axis_index("c")
        sid = lax.axis_index("s")
        wid = sid * 2 + cid
        idxb = (idx0_v, idx1_v)
        sems = (sem0, sem1)
        lane_f = [jnp.full((16,), f, jnp.int32) for f in range(FPT)]

        pltpu.sync_copy(t_hbm.at[pl.ds(wid * FPT, FPT)], tab_v)

        def start(seg, buf):
            pltpu.async_copy(adj_hbm.at[seg], idxb[buf], sems[buf])

        def hop(src_v, dst_v):
            def process(seg, b):
                pltpu.make_async_copy(adj_hbm.at[0], idxb[b],
                                      sems[b]).wait()

                @pl.loop(0, BLOCKS, unroll=5)
                def block(nb):
                    col = seg * SEG + nb * 16
                    acc = [jnp.zeros((16,), jnp.float32)
                           for _ in range(FPT)]
                    for j in range(DEG):
                        nbr = idxb[b][j, pl.ds(nb * 16, 16)]
                        for f in range(FPT):
                            acc[f] = acc[f] + plsc.load_gather(
                                src_v, [lane_f[f], nbr])
                    for f in range(FPT):
                        dst_v[f, pl.ds(col, 16)] = acc[f] * (1.0 / DEG)

            start(0, 0)

            @pl.loop(0, SEGS - 1, step=2)
            def seg_pair(g):
                for b in (0, 1):
                    gg = g + b

                    @pl.when(gg + 1 < SEGS)
                    def _():
                        start(gg + 1, 1 - b)

                    process(gg, b)

            process(SEGS - 1, (SEGS - 1) % 2)

        hop(tab_v, h1_v)
        hop(h1_v, tab_v)
        pltpu.sync_copy(tab_v, out_hbm.at[pl.ds(wid * FPT, FPT)])

    return pl.kernel(
        body,
        out_type=jax.ShapeDtypeStruct((KDIM, N), jnp.float32),
        mesh=plsc.VectorSubcoreMesh(core_axis_name="c", subcore_axis_name="s"),
        scratch_types=[
            pltpu.VMEM((FPT, N), jnp.float32),
            pltpu.VMEM((FPT, N), jnp.float32),
            pltpu.VMEM((DEG, SEG), jnp.int32),
            pltpu.VMEM((DEG, SEG), jnp.int32),
            pltpu.SemaphoreType.DMA,
            pltpu.SemaphoreType.DMA,
        ],
        compiler_params=pltpu.CompilerParams(use_tc_tiling_on_sc=False,
                                             needs_layout_passes=False),
    )


_layer = _make_layer()


def _ew1_body(d_ref, o_ref):
    e = jnp.exp(d_ref[...])
    s = jnp.sum(e, axis=1, keepdims=True)
    o_ref[...] = (e / jnp.where(s == 0.0, 1.0, s)).T


def _ew1(d):
    return pl.pallas_call(
        _ew1_body,
        out_shape=jax.ShapeDtypeStruct((KDIM, N), jnp.float32),
    )(d)


def _ew2_body(d_ref, o_ref):
    o_ref[...] = jnp.log(jnp.clip(d_ref[...], 1e-9, None)).T


def _ew2(d):
    return pl.pallas_call(
        _ew2_body,
        out_shape=jax.ShapeDtypeStruct((N, KDIM), jnp.float32),
    )(d)


def _fin_body(q1_ref, q2_ref, b_ref, o_ref):
    o_ref[...] = jnp.dot(
        q1_ref[...] + q2_ref[...],
        b_ref[...],
        precision=lax.Precision.HIGHEST,
        preferred_element_type=jnp.float32,
    )


_FIN_BLOCK = 1000


def _final(q1, q2, b2):
    return pl.pallas_call(
        _fin_body,
        grid=(N // _FIN_BLOCK,),
        in_specs=[
            pl.BlockSpec((_FIN_BLOCK, KDIM), lambda i: (i, 0)),
            pl.BlockSpec((_FIN_BLOCK, KDIM), lambda i: (i, 0)),
            pl.BlockSpec((KDIM, 128), lambda i: (0, 0)),
        ],
        out_specs=pl.BlockSpec((_FIN_BLOCK, 128), lambda i: (i, 0)),
        out_shape=jax.ShapeDtypeStruct((N, 128), jnp.float32),
    )(q1, q2, b2)


def kernel(x, adj_index):
    adj = adj_index.astype(jnp.int32)
    adj_r = adj.reshape(SEGS, SEG, DEG).transpose(0, 2, 1)

    U, S, Vt = jnp.linalg.svd(x, full_matrices=False)
    dis = U[:, :N_COMPONENT]
    s64 = S[:N_COMPONENT]
    base = Vt[:N_COMPONENT, :]

    qs = []
    for _ in range(LAYER_L):
        t = _ew1(dis)
        h2 = _layer(adj_r, t)
        a = _ew2(h2)
        q, _ = jnp.linalg.qr(a)
        dis = q
        qs.append(q)

    b2 = (0.5 * s64)[:, None] * base
    return _final(qs[0], qs[1], b2)


# speedup vs baseline: 1.0008x; 1.0008x over previous
import jax
import jax.numpy as jnp
from jax import lax
from jax.experimental import pallas as pl
from jax.experimental.pallas import tpu as pltpu
from jax.experimental.pallas import tpu_sc as plsc

N_COMPONENT = 64
H_HOP = 2
LAYER_L = 2
DEG = 32

N = 10000
KDIM = 64

NW = 32
FPT = KDIM // NW
SEG = 400
SEGS = N // SEG
BLOCKS = SEG // 16


def _make_layer():
    def body(adj_hbm, t_hbm, out_hbm, tab_v, h1_v, idx0_v, idx1_v,
             sem0, sem1):
        cid = lax.axis_index("c")
        sid = lax.axis_index("s")
        wid = sid * 2 + cid
        idxb = (idx0_v, idx1_v)
        sems = (sem0, sem1)
        lane_f = [jnp.full((16,), f, jnp.int32) for f in range(FPT)]

        pltpu.sync_copy(t_hbm.at[pl.ds(wid * FPT, FPT)], tab_v)

        def start(seg, buf):
            pltpu.async_copy(adj_hbm.at[seg], idxb[buf], sems[buf])

        def hop(src_v, dst_v):
            def process(seg, b):
                pltpu.make_async_copy(adj_hbm.at[0], idxb[b],
                                      sems[b]).wait()

                @pl.loop(0, BLOCKS)
                def block(nb):
                    col = seg * SEG + nb * 16
                    acc = [jnp.zeros((16,), jnp.float32)
                           for _ in range(FPT)]
                    for j in range(DEG):
                        nbr = idxb[b][j, pl.ds(nb * 16, 16)]
                        for f in range(FPT):
                            acc[f] = acc[f] + plsc.load_gather(
                                src_v, [lane_f[f], nbr])
                    for f in range(FPT):
                        dst_v[f, pl.ds(col, 16)] = acc[f] * (1.0 / DEG)

            start(0, 0)

            @pl.loop(0, SEGS - 1, step=2)
            def seg_pair(g):
                for b in (0, 1):
                    gg = g + b

                    @pl.when(gg + 1 < SEGS)
                    def _():
                        start(gg + 1, 1 - b)

                    process(gg, b)

            process(SEGS - 1, (SEGS - 1) % 2)

        hop(tab_v, h1_v)
        hop(h1_v, tab_v)
        pltpu.sync_copy(tab_v, out_hbm.at[pl.ds(wid * FPT, FPT)])

    return pl.kernel(
        body,
        out_type=jax.ShapeDtypeStruct((KDIM, N), jnp.float32),
        mesh=plsc.VectorSubcoreMesh(core_axis_name="c", subcore_axis_name="s"),
        scratch_types=[
            pltpu.VMEM((FPT, N), jnp.float32),
            pltpu.VMEM((FPT, N), jnp.float32),
            pltpu.VMEM((DEG, SEG), jnp.int32),
            pltpu.VMEM((DEG, SEG), jnp.int32),
            pltpu.SemaphoreType.DMA,
            pltpu.SemaphoreType.DMA,
        ],
        compiler_params=pltpu.CompilerParams(use_tc_tiling_on_sc=False,
                                             needs_layout_passes=False),
    )


_layer = _make_layer()


def _ew1_body(d_ref, o_ref):
    e = jnp.exp(d_ref[...])
    s = jnp.sum(e, axis=1, keepdims=True)
    o_ref[...] = (e / jnp.where(s == 0.0, 1.0, s)).T


def _ew1(d):
    return pl.pallas_call(
        _ew1_body,
        out_shape=jax.ShapeDtypeStruct((KDIM, N), jnp.float32),
    )(d)


def _ew2_body(d_ref, o_ref):
    o_ref[...] = jnp.log(jnp.clip(d_ref[...], 1e-9, None)).T


def _ew2(d):
    return pl.pallas_call(
        _ew2_body,
        out_shape=jax.ShapeDtypeStruct((N, KDIM), jnp.float32),
    )(d)


def _fin_body(q1_ref, q2_ref, b_ref, o_ref):
    o_ref[...] = jnp.dot(
        q1_ref[...] + q2_ref[...],
        b_ref[...],
        precision=lax.Precision.HIGHEST,
        preferred_element_type=jnp.float32,
    )


_FIN_BLOCK = 1000


def _final(q1, q2, b2):
    return pl.pallas_call(
        _fin_body,
        grid=(N // _FIN_BLOCK,),
        in_specs=[
            pl.BlockSpec((_FIN_BLOCK, KDIM), lambda i: (i, 0)),
            pl.BlockSpec((_FIN_BLOCK, KDIM), lambda i: (i, 0)),
            pl.BlockSpec((KDIM, 128), lambda i: (0, 0)),
        ],
        out_specs=pl.BlockSpec((_FIN_BLOCK, 128), lambda i: (i, 0)),
        out_shape=jax.ShapeDtypeStruct((N, 128), jnp.float32),
    )(q1, q2, b2)


def kernel(x, adj_index):
    adj = adj_index.astype(jnp.int32)
    adj_r = adj.reshape(SEGS, SEG, DEG).transpose(0, 2, 1)

    U, S, Vt = jnp.linalg.svd(x, full_matrices=False)
    dis = U[:, :N_COMPONENT]
    s64 = S[:N_COMPONENT]
    base = Vt[:N_COMPONENT, :]

    qs = []
    for _ in range(LAYER_L):
        t = _ew1(dis)
        h2 = _layer(adj_r, t)
        a = _ew2(h2)
        q, _ = jnp.linalg.qr(a)
        dis = q
        qs.append(q)

    b2 = (0.5 * s64)[:, None] * base
    return _final(qs[0], qs[1], b2)
